# Initial kernel scaffold; baseline (speedup 1.0000x reference)
import jax
import jax.numpy as jnp
from jax.experimental import pallas as pl


def kernel(x, edge_index, W1, b1, W2, b2, W3, b3, W4, b4, W5, b5, Wl, bl):
    raise NotImplementedError("write your pallas kernel here")


# speedup vs baseline: 5.4600x; 5.4600x over previous
import functools

import jax
import jax.numpy as jnp
from jax import lax
from jax.experimental import pallas as pl
from jax.experimental.pallas import tpu as pltpu
from jax.experimental.pallas import tpu_sc as plsc

N = 50000
NP = 51200
E = 800000
EP = 819200
NC = 2
NS = 16
STRIPE = NP // NS
CW = 16
BN = 512
GRID = NP // BN

_mesh = lambda: plsc.VectorSubcoreMesh(
    core_axis_name="c", subcore_axis_name="s", num_cores=NC, num_subcores=NS)
_sc_params = pltpu.CompilerParams(use_tc_tiling_on_sc=False)


@functools.partial(
    pl.kernel,
    out_type=jax.ShapeDtypeStruct((NC, NP, CW), jnp.float32),
    mesh=_mesh(),
    scratch_types=[
        pltpu.VMEM((8, 128), jnp.int32),
        pltpu.VMEM((128, CW), jnp.float32),
        pltpu.VMEM_SHARED((NP, CW), jnp.float32),
    ],
    compiler_params=_sc_params,
)
def _sc_deg(dst_hbm, zeros_hbm, ones_hbm, out_hbm, idx_v, ones_v, acc_sh):
    c = lax.axis_index("c")
    s = lax.axis_index("s")
    row0 = s * STRIPE
    pltpu.sync_copy(ones_hbm, ones_v)
    pltpu.sync_copy(zeros_hbm.at[pl.ds(row0, STRIPE)],
                    acc_sh.at[pl.ds(row0, STRIPE)])
    plsc.subcore_barrier()

    def body(g, carry):
        gg = c * 25 + g
        pltpu.sync_copy(dst_hbm.at[s].at[gg], idx_v)
        for j in range(8):
            pltpu.sync_copy(ones_v, acc_sh.at[idx_v.at[j]], add=True)
        return carry

    lax.fori_loop(0, 25, body, 0)
    plsc.subcore_barrier()
    pltpu.sync_copy(acc_sh.at[pl.ds(row0, STRIPE)],
                    out_hbm.at[c].at[pl.ds(row0, STRIPE)])


def _make_scatter(ch):
    @functools.partial(
        pl.kernel,
        out_type=jax.ShapeDtypeStruct((ch, NP, CW), jnp.float32),
        mesh=_mesh(),
        scratch_types=[
            pltpu.VMEM((8, 128), jnp.int32),
            pltpu.VMEM((8, 128), jnp.int32),
            pltpu.VMEM((8, 128, CW), jnp.float32),
            pltpu.VMEM_SHARED((NP, CW), jnp.float32),
            pltpu.SemaphoreType.DMA,
        ],
        compiler_params=_sc_params,
    )
    def _sc_scatter(src_hbm, dst_hbm, ych_hbm, out_hbm,
                    idx_s, idx_d, rows, acc_sh, sem):
        c = lax.axis_index("c")
        s = lax.axis_index("s")
        row0 = s * STRIPE
        for p in range(ch // 2):
            chunk = c * (ch // 2) + p
            pltpu.sync_copy(ych_hbm.at[chunk].at[pl.ds(row0, STRIPE)],
                            acc_sh.at[pl.ds(row0, STRIPE)])
            plsc.subcore_barrier()

            def body(g, carry):
                pltpu.sync_copy(src_hbm.at[s].at[g], idx_s)
                pltpu.sync_copy(dst_hbm.at[s].at[g], idx_d)
                descs = [
                    pltpu.async_copy(ych_hbm.at[chunk].at[idx_s.at[j]],
                                     rows.at[j], sem)
                    for j in range(8)
                ]
                for d in descs:
                    d.wait()
                for j in range(8):
                    pltpu.sync_copy(rows.at[j], acc_sh.at[idx_d.at[j]],
                                    add=True)
                return carry

            lax.fori_loop(0, 50, body, 0)
            plsc.subcore_barrier()
            pltpu.sync_copy(acc_sh.at[pl.ds(row0, STRIPE)],
                            out_hbm.at[chunk].at[pl.ds(row0, STRIPE)])
            plsc.subcore_barrier()

    return _sc_scatter


_sc_scatter2 = _make_scatter(2)
_sc_scatter8 = _make_scatter(8)


def _tc1_body(x_ref, deg_ref, w_ref, ych_ref, dinv_ref):
    deg = deg_ref[0, :, 0:1] + deg_ref[1, :, 0:1] + 1.0
    dinv = lax.rsqrt(deg)
    x = x_ref[...]
    xw = x[:, 0:1] * w_ref[0:1, :] + x[:, 1:2] * w_ref[1:2, :]
    y = xw * dinv
    for ci in range(2):
        ych_ref[ci] = y[:, CW * ci:CW * ci + CW]
    dinv_ref[...] = dinv


def _tcmid_body(nch_in, acc_ref, dinv_ref, b_ref, w_ref, ych_ref):
    dinv = dinv_ref[...]
    hcat = jnp.concatenate([acc_ref[i] for i in range(nch_in)], axis=1)
    h = jnp.maximum(dinv * hcat + b_ref[...], 0.0)
    y = jnp.dot(h, w_ref[...], preferred_element_type=jnp.float32) * dinv
    for ci in range(8):
        ych_ref[ci] = y[:, CW * ci:CW * ci + CW]


def _tcfinal_body(acc_ref, dinv_ref, b_ref, wlt_ref, bl_ref, o_ref):
    dinv = dinv_ref[...]
    hcat = jnp.concatenate([acc_ref[i] for i in range(8)], axis=1)
    h = jnp.maximum(dinv * hcat + b_ref[...], 0.0)
    o = jnp.sum(h * wlt_ref[...], axis=1, keepdims=True) + bl_ref[0, 0]
    o_ref[...] = jnp.where(o > 0, o, 0.01 * o)


def _row_spec(width):
    return pl.BlockSpec((BN, width), lambda i: (i, 0))


def _chunk_spec(nch, width=CW):
    return pl.BlockSpec((nch, BN, width), lambda i: (0, i, 0))


def _full_spec(a, b):
    return pl.BlockSpec((a, b), lambda i: (0, 0))


_tc1 = pl.pallas_call(
    _tc1_body,
    grid=(GRID,),
    in_specs=[_row_spec(2), _chunk_spec(2), _full_spec(2, 32)],
    out_specs=[_chunk_spec(2), _row_spec(1)],
    out_shape=[jax.ShapeDtypeStruct((2, NP, CW), jnp.float32),
               jax.ShapeDtypeStruct((NP, 1), jnp.float32)],
)

_tc2 = pl.pallas_call(
    functools.partial(_tcmid_body, 2),
    grid=(GRID,),
    in_specs=[_chunk_spec(2), _row_spec(1), _full_spec(1, 32),
              _full_spec(32, 128)],
    out_specs=_chunk_spec(8),
    out_shape=jax.ShapeDtypeStruct((8, NP, CW), jnp.float32),
)

_tcmid = pl.pallas_call(
    functools.partial(_tcmid_body, 8),
    grid=(GRID,),
    in_specs=[_chunk_spec(8), _row_spec(1), _full_spec(1, 128),
              _full_spec(128, 128)],
    out_specs=_chunk_spec(8),
    out_shape=jax.ShapeDtypeStruct((8, NP, CW), jnp.float32),
)

_tcfinal = pl.pallas_call(
    _tcfinal_body,
    grid=(GRID,),
    in_specs=[_chunk_spec(8), _row_spec(1), _full_spec(1, 128),
              _full_spec(1, 128), _full_spec(1, 1)],
    out_specs=_row_spec(1),
    out_shape=jax.ShapeDtypeStruct((NP, 1), jnp.float32),
)


def kernel(x, edge_index, W1, b1, W2, b2, W3, b3, W4, b4, W5, b5, Wl, bl):
    f32 = jnp.float32
    src = edge_index[0].astype(jnp.int32)
    dst = edge_index[1].astype(jnp.int32)
    src_p = jnp.concatenate([src, jnp.zeros((EP - E,), jnp.int32)])
    dst_p = jnp.concatenate([dst, jnp.full((EP - E,), N, jnp.int32)])
    src_r = src_p.reshape(NS, 50, 8, 128)
    dst_r = dst_p.reshape(NS, 50, 8, 128)

    x_pad = jnp.zeros((NP, 2), f32).at[:N].set(x)
    zeros16 = jnp.zeros((NP, CW), f32)
    ones16 = jnp.ones((128, CW), f32)

    degp = _sc_deg(dst_r, zeros16, ones16)
    ych, dinv = _tc1(x_pad, degp, W1)
    acc = _sc_scatter2(src_r, dst_r, ych)
    ych = _tc2(acc, dinv, b1.reshape(1, 32), W2)
    for b, w in ((b2, W3), (b3, W4), (b4, W5)):
        acc = _sc_scatter8(src_r, dst_r, ych)
        ych = _tcmid(acc, dinv, b.reshape(1, 128), w)
    acc5 = _sc_scatter8(src_r, dst_r, ych)
    o = _tcfinal(acc5, dinv, b5.reshape(1, 128), Wl.T.reshape(1, 128),
                 bl.reshape(1, 1))
    return o[:N]


# speedup vs baseline: 5.5900x; 1.0238x over previous
import functools

import jax
import jax.numpy as jnp
from jax import lax
from jax.experimental import pallas as pl
from jax.experimental.pallas import tpu as pltpu
from jax.experimental.pallas import tpu_sc as plsc

N = 50000
NP = 51200
E = 800000
EP = 819200
NC = 2
NS = 16
STRIPE = NP // NS
CW = 16
BN = 512
GRID = NP // BN

_mesh = lambda: plsc.VectorSubcoreMesh(
    core_axis_name="c", subcore_axis_name="s", num_cores=NC, num_subcores=NS)
_sc_params = pltpu.CompilerParams(use_tc_tiling_on_sc=False,
                                  needs_layout_passes=False)


@functools.partial(
    pl.kernel,
    out_type=jax.ShapeDtypeStruct((NC, NP, CW), jnp.float32),
    mesh=_mesh(),
    scratch_types=[
        pltpu.VMEM((8, 128), jnp.int32),
        pltpu.VMEM((128, CW), jnp.float32),
        pltpu.VMEM_SHARED((NP, CW), jnp.float32),
    ],
    compiler_params=_sc_params,
)
def _sc_deg(dst_hbm, zeros_hbm, ones_hbm, out_hbm, idx_v, ones_v, acc_sh):
    c = lax.axis_index("c")
    s = lax.axis_index("s")
    row0 = s * STRIPE
    pltpu.sync_copy(ones_hbm, ones_v)
    pltpu.sync_copy(zeros_hbm.at[pl.ds(row0, STRIPE)],
                    acc_sh.at[pl.ds(row0, STRIPE)])
    plsc.subcore_barrier()

    def body(g, carry):
        gg = c * 25 + g
        pltpu.sync_copy(dst_hbm.at[s].at[gg], idx_v)
        for j in range(8):
            pltpu.sync_copy(ones_v, acc_sh.at[idx_v.at[j]], add=True)
        return carry

    lax.fori_loop(0, 25, body, 0)
    plsc.subcore_barrier()
    pltpu.sync_copy(acc_sh.at[pl.ds(row0, STRIPE)],
                    out_hbm.at[c].at[pl.ds(row0, STRIPE)])


@functools.partial(
    pl.kernel,
    out_type=jax.ShapeDtypeStruct((NS, 50, 8, 128), jnp.float32),
    mesh=_mesh(),
    scratch_types=[
        pltpu.VMEM((NP,), jnp.float32),
        pltpu.VMEM((8, 128), jnp.int32),
        pltpu.VMEM((8, 128), jnp.int32),
        pltpu.VMEM((8, 128), jnp.float32),
    ],
    compiler_params=_sc_params,
)
def _sc_norm(src_hbm, dst_hbm, dinv_hbm, out_hbm, dinv_v, idx_s, idx_d, nv):
    c = lax.axis_index("c")
    s = lax.axis_index("s")
    pltpu.sync_copy(dinv_hbm, dinv_v)

    def body(g, carry):
        gg = c * 25 + g
        pltpu.sync_copy(src_hbm.at[s].at[gg], idx_s)
        pltpu.sync_copy(dst_hbm.at[s].at[gg], idx_d)
        for j in range(8):
            def inner(k, carry2):
                iv_s = idx_s[j, pl.ds(k * 16, 16)]
                iv_d = idx_d[j, pl.ds(k * 16, 16)]
                a = plsc.load_gather(dinv_v, [iv_s])
                b = plsc.load_gather(dinv_v, [iv_d])
                nv[j, pl.ds(k * 16, 16)] = a * b
                return carry2

            lax.fori_loop(0, 8, inner, 0)
        pltpu.sync_copy(nv, out_hbm.at[s].at[gg])
        return carry

    lax.fori_loop(0, 25, body, 0)


def _make_scatter(ch):
    @functools.partial(
        pl.kernel,
        out_type=jax.ShapeDtypeStruct((ch, NP, CW), jnp.float32),
        mesh=_mesh(),
        scratch_types=[
            pltpu.VMEM((2, 8, 128), jnp.int32),
            pltpu.VMEM((2, 8, 128), jnp.int32),
            pltpu.VMEM((2, 8, 128), jnp.float32),
            pltpu.VMEM((2, 8, 128, CW), jnp.float32),
            pltpu.VMEM_SHARED((NP, CW), jnp.float32),
            pltpu.SemaphoreType.DMA,
            pltpu.SemaphoreType.DMA,
        ],
        compiler_params=_sc_params,
    )
    def _sc_scatter(src_hbm, dst_hbm, norm_hbm, xw_hbm, yself_hbm, out_hbm,
                    idx_s, idx_d, nrm, rows, acc_sh, sem0, sem1):
        c = lax.axis_index("c")
        s = lax.axis_index("s")
        row0 = s * STRIPE
        sems = (sem0, sem1)

        for p in range(ch // 2):
            chunk = c * (ch // 2) + p
            pltpu.sync_copy(yself_hbm.at[chunk].at[pl.ds(row0, STRIPE)],
                            acc_sh.at[pl.ds(row0, STRIPE)])
            plsc.subcore_barrier()

            def fire(g, bi):
                pltpu.sync_copy(src_hbm.at[s].at[g], idx_s.at[bi])
                pltpu.sync_copy(dst_hbm.at[s].at[g], idx_d.at[bi])
                pltpu.sync_copy(norm_hbm.at[s].at[g], nrm.at[bi])
                for j in range(8):
                    pltpu.async_copy(xw_hbm.at[chunk].at[idx_s.at[bi].at[j]],
                                     rows.at[bi].at[j], sems[bi])

            def drain_scale_scatter(bi):
                for j in range(8):
                    pltpu.make_async_copy(
                        xw_hbm.at[chunk].at[idx_s.at[bi].at[j]],
                        rows.at[bi].at[j], sems[bi]).wait()
                for j in range(8):
                    def scale16(m, carry2):
                        nvec = nrm[bi, j, pl.ds(m * 16, 16)]
                        for t in range(16):
                            r = m * 16 + t
                            rows[bi, j, r] = rows[bi, j, r] * nvec[t]
                        return carry2

                    lax.fori_loop(0, 8, scale16, 0)
                for j in range(8):
                    pltpu.sync_copy(rows.at[bi].at[j],
                                    acc_sh.at[idx_d.at[bi].at[j]], add=True)

            fire(0, 0)

            def body(k, carry):
                fire(2 * k + 1, 1)
                drain_scale_scatter(0)

                @pl.when(k < 24)
                def _():
                    fire(2 * k + 2, 0)

                drain_scale_scatter(1)
                return carry

            lax.fori_loop(0, 25, body, 0)
            plsc.subcore_barrier()
            pltpu.sync_copy(acc_sh.at[pl.ds(row0, STRIPE)],
                            out_hbm.at[chunk].at[pl.ds(row0, STRIPE)])
            plsc.subcore_barrier()

    return _sc_scatter


_sc_scatter2 = _make_scatter(2)
_sc_scatter8 = _make_scatter(8)


def _tc1_body(x_ref, dinv_ref, w_ref, xwch_ref, ysch_ref):
    dinv = dinv_ref[...]
    xw = jnp.dot(x_ref[...], w_ref[...], preferred_element_type=jnp.float32)
    ys = xw * (dinv * dinv)
    for ci in range(2):
        xwch_ref[ci] = xw[:, CW * ci:CW * ci + CW]
        ysch_ref[ci] = ys[:, CW * ci:CW * ci + CW]


def _tcmid_body(nch_in, acc_ref, dinv_ref, b_ref, w_ref, xwch_ref, ysch_ref):
    dinv = dinv_ref[...]
    hcat = jnp.concatenate([acc_ref[i] for i in range(nch_in)], axis=1)
    h = jnp.maximum(hcat + b_ref[...], 0.0)
    xw = jnp.dot(h, w_ref[...], preferred_element_type=jnp.float32)
    ys = xw * (dinv * dinv)
    for ci in range(8):
        xwch_ref[ci] = xw[:, CW * ci:CW * ci + CW]
        ysch_ref[ci] = ys[:, CW * ci:CW * ci + CW]


def _tcfinal_body(acc_ref, b_ref, wlt_ref, bl_ref, o_ref):
    hcat = jnp.concatenate([acc_ref[i] for i in range(8)], axis=1)
    h = jnp.maximum(hcat + b_ref[...], 0.0)
    o = jnp.sum(h * wlt_ref[...], axis=1, keepdims=True) + bl_ref[0, 0]
    o_ref[...] = jnp.where(o > 0, o, 0.01 * o)


def _row_spec(width):
    return pl.BlockSpec((BN, width), lambda i: (i, 0))


def _chunk_spec(nch, width=CW):
    return pl.BlockSpec((nch, BN, width), lambda i: (0, i, 0))


def _full_spec(a, b):
    return pl.BlockSpec((a, b), lambda i: (0, 0))


_tc1 = pl.pallas_call(
    _tc1_body,
    grid=(GRID,),
    in_specs=[_row_spec(2), _row_spec(1), _full_spec(2, 32)],
    out_specs=[_chunk_spec(2), _chunk_spec(2)],
    out_shape=[jax.ShapeDtypeStruct((2, NP, CW), jnp.float32),
               jax.ShapeDtypeStruct((2, NP, CW), jnp.float32)],
)

_tc2 = pl.pallas_call(
    functools.partial(_tcmid_body, 2),
    grid=(GRID,),
    in_specs=[_chunk_spec(2), _row_spec(1), _full_spec(1, 32),
              _full_spec(32, 128)],
    out_specs=[_chunk_spec(8), _chunk_spec(8)],
    out_shape=[jax.ShapeDtypeStruct((8, NP, CW), jnp.float32),
               jax.ShapeDtypeStruct((8, NP, CW), jnp.float32)],
)

_tcmid = pl.pallas_call(
    functools.partial(_tcmid_body, 8),
    grid=(GRID,),
    in_specs=[_chunk_spec(8), _row_spec(1), _full_spec(1, 128),
              _full_spec(128, 128)],
    out_specs=[_chunk_spec(8), _chunk_spec(8)],
    out_shape=[jax.ShapeDtypeStruct((8, NP, CW), jnp.float32),
               jax.ShapeDtypeStruct((8, NP, CW), jnp.float32)],
)

_tcfinal = pl.pallas_call(
    _tcfinal_body,
    grid=(GRID,),
    in_specs=[_chunk_spec(8), _full_spec(1, 128),
              _full_spec(1, 128), _full_spec(1, 1)],
    out_specs=_row_spec(1),
    out_shape=jax.ShapeDtypeStruct((NP, 1), jnp.float32),
)


def kernel(x, edge_index, W1, b1, W2, b2, W3, b3, W4, b4, W5, b5, Wl, bl):
    f32 = jnp.float32
    src = edge_index[0].astype(jnp.int32)
    dst = edge_index[1].astype(jnp.int32)
    src_p = jnp.concatenate([src, jnp.zeros((EP - E,), jnp.int32)])
    dst_p = jnp.concatenate([dst, jnp.full((EP - E,), N, jnp.int32)])
    src_r = src_p.reshape(NS, 50, 8, 128)
    dst_r = dst_p.reshape(NS, 50, 8, 128)

    x_pad = jnp.zeros((NP, 2), f32).at[:N].set(x)
    zeros16 = jnp.zeros((NP, CW), f32)
    ones16 = jnp.ones((128, CW), f32)

    degp = _sc_deg(dst_r, zeros16, ones16)
    deg = degp[0, :, 0:1] + degp[1, :, 0:1] + 1.0
    dinv = jnp.where(deg > 0, 1.0 / jnp.sqrt(deg), 0.0)
    xwch, ysch = _tc1(x_pad, dinv, W1)
    norm_r = _sc_norm(src_r, dst_r, dinv[:, 0])
    acc = _sc_scatter2(src_r, dst_r, norm_r, xwch, ysch)
    xwch, ysch = _tc2(acc, dinv, b1.reshape(1, 32), W2)
    for b, w in ((b2, W3), (b3, W4), (b4, W5)):
        acc = _sc_scatter8(src_r, dst_r, norm_r, xwch, ysch)
        xwch, ysch = _tcmid(acc, dinv, b.reshape(1, 128), w)
    acc5 = _sc_scatter8(src_r, dst_r, norm_r, xwch, ysch)
    o = _tcfinal(acc5, b5.reshape(1, 128), Wl.T.reshape(1, 128),
                 bl.reshape(1, 1))
    return o[:N]
